# baseline (speedup 1.0000x reference)
import jax
import jax.numpy as jnp
from jax import lax
from jax.experimental import pallas as pl
from jax.experimental.pallas import tpu as pltpu
from jax.experimental.pallas import tpu_sc as plsc

_B = 16384
_F = 64
_L = 16
_NC = 2
_NS = 16
_NW = _NC * _NS
_PER_W = _B // _NW
_CHUNK = 128
_NCH = _PER_W // _CHUNK
_NBLK = _PER_W // _L

_mesh = plsc.VectorSubcoreMesh(core_axis_name="c", subcore_axis_name="s")


def _sc_body(x_hbm, lab_hbm, cen_hbm, out_hbm,
             idx_v, par_v, x_v, rows_v, acc_v, sem):
    wid = lax.axis_index("s") * _NC + lax.axis_index("c")
    base = wid * _PER_W

    pltpu.sync_copy(lab_hbm.at[pl.ds(base, _PER_W)], idx_v)

    @pl.loop(0, _PER_W, step=_L)
    def _(i):
        lv = idx_v[pl.ds(i, _L)]
        idx_v[pl.ds(i, _L)] = lax.shift_right_logical(lv, 1)
        par_v[pl.ds(i, _L)] = lax.shift_left(jnp.bitwise_and(lv, 1), 6)

    copies = [
        pltpu.async_copy(
            cen_hbm.at[idx_v.at[pl.ds(j * _CHUNK, _CHUNK)]],
            rows_v.at[pl.ds(j * _CHUNK, _CHUNK)],
            sem,
        )
        for j in range(_NCH)
    ]
    pltpu.sync_copy(x_hbm.at[pl.ds(base * _F, _PER_W * _F)], x_v)
    for c in copies:
        c.wait()

    lane = lax.iota(jnp.int32, _L)

    def blk_body(rb, acc):
        rows16 = rb * _L + lane
        coff = par_v[pl.ds(rb * _L, _L)]
        xbase = rows16 * _F

        def col_body(f, acc):
            fv = jnp.full((_L,), f, jnp.int32)
            xv = plsc.load_gather(x_v, [xbase + fv])
            cv = plsc.load_gather(rows_v, [rows16, coff + fv])
            d = xv - cv
            return acc + d * d

        return lax.fori_loop(0, _F, col_body, acc)

    acc = lax.fori_loop(0, _NBLK, blk_body, jnp.zeros((_L,), jnp.float32))
    acc_v[...] = acc * (0.5 / _B)
    pltpu.sync_copy(acc_v, out_hbm.at[wid])


@jax.jit
def kernel(x, labels, centers):
    labels = labels.astype(jnp.int32)
    x = x.reshape(-1)
    centers2 = centers.reshape(centers.shape[0] // 2, 2 * centers.shape[1])
    run = pl.kernel(
        _sc_body,
        out_type=jax.ShapeDtypeStruct((_NW, _L), jnp.float32),
        mesh=_mesh,
        compiler_params=pltpu.CompilerParams(needs_layout_passes=False),
        scratch_types=[
            pltpu.VMEM((_PER_W,), jnp.int32),
            pltpu.VMEM((_PER_W,), jnp.int32),
            pltpu.VMEM((_PER_W * _F,), jnp.float32),
            pltpu.VMEM((_PER_W, 2 * _F), jnp.float32),
            pltpu.VMEM((_L,), jnp.float32),
            pltpu.SemaphoreType.DMA,
        ],
    )
    partials = run(x, labels, centers2)
    return jnp.sum(partials)


# speedup vs baseline: 2.1530x; 2.1530x over previous
import jax
import jax.numpy as jnp
from jax import lax
from jax.experimental import pallas as pl
from jax.experimental.pallas import tpu as pltpu
from jax.experimental.pallas import tpu_sc as plsc

_B = 16384
_F = 64
_L = 16
_NC = 2
_NS = 16
_NW = _NC * _NS
_PER_W = _B // _NW
_PH = 256
_NPH = _PER_W // _PH

_mesh = plsc.VectorSubcoreMesh(core_axis_name="c", subcore_axis_name="s")


def _scalar(vec, j):
    return lax.squeeze(lax.slice_in_dim(vec, j, j + 1), (0,))


def _sc_body(x_hbm, lab_hbm, cen_hbm, out_hbm,
             idx_v, x_v, rows_v, acc_v, sem, xsem):
    wid = lax.axis_index("s") * _NC + lax.axis_index("c")
    base = wid * _PER_W

    pltpu.sync_copy(lab_hbm.at[pl.ds(base, _PER_W)], idx_v)

    acc = jnp.zeros((_L,), jnp.float32)
    for p in range(_NPH):
        @pl.loop(0, _PH // _L)
        def _(ch):
            lv = idx_v[pl.ds(p * _PH + ch * _L, _L)]
            for j in range(_L):
                l = _scalar(lv, j)
                t = lax.shift_right_logical(l, 3)
                q = jnp.bitwise_and(l, 7)
                pltpu.async_copy(
                    cen_hbm.at[t, q], rows_v.at[ch * _L + j], sem
                )

        pltpu.async_copy(
            x_hbm.at[pl.ds(base + p * _PH, _PH)], x_v, xsem
        ).wait()

        @pl.loop(0, _PH)
        def _(r):
            pltpu.make_async_copy(cen_hbm.at[0, 0], rows_v.at[0], sem).wait()

        def row_body(r, acc):
            for cc in range(_F // _L):
                xv = x_v[r, pl.ds(cc * _L, _L)]
                cv = rows_v[r, pl.ds(cc * _L, _L)]
                d = xv - cv
                acc = acc + d * d
            return acc

        acc = lax.fori_loop(0, _PH, row_body, acc)

    acc_v[...] = acc * (0.5 / _B)
    pltpu.sync_copy(acc_v, out_hbm.at[wid])


@jax.jit
def kernel(x, labels, centers):
    labels = labels.astype(jnp.int32)
    centers = centers.reshape(-1, 8, _F)
    run = pl.kernel(
        _sc_body,
        out_type=jax.ShapeDtypeStruct((_NW, _L), jnp.float32),
        mesh=_mesh,
        compiler_params=pltpu.CompilerParams(needs_layout_passes=False),
        scratch_types=[
            pltpu.VMEM((_PER_W,), jnp.int32),
            pltpu.VMEM((_PH, _F), jnp.float32),
            pltpu.VMEM((_PH, _F), jnp.float32),
            pltpu.VMEM((_L,), jnp.float32),
            pltpu.SemaphoreType.DMA,
            pltpu.SemaphoreType.DMA,
        ],
    )
    partials = run(x, labels, centers)
    return jnp.sum(partials)
